# baseline (speedup 1.0000x reference)
import functools

import jax
import jax.numpy as jnp
from jax import lax
from jax.experimental import pallas as pl
from jax.experimental.pallas import tpu as pltpu
from jax.experimental.pallas import tpu_sc as plsc

N = 16384
C = 1000
L = 16
NC = 2
NS = 16
NW = NC * NS
RPW = N // NW
G = 128
NG = RPW // G


def _nll_body(lp_hbm, tgt_hbm, out_hbm,
              tgt_v, idx0, idx1, idx2, idx3, got0, got1, got2, got3,
              my_v, shared, sem):
    idx_bufs = (idx0, idx1, idx2, idx3)
    got_bufs = (got0, got1, got2, got3)
    cid = lax.axis_index("c")
    sid = lax.axis_index("s")
    wid = cid * NS + sid
    base = wid * RPW

    pltpu.sync_copy(tgt_hbm.at[pl.ds(base, RPW)], tgt_v)

    for g in range(NG):
        for k in range(G // L):
            off = g * G + k * L
            t = tgt_v[pl.ds(off, L)]
            rows = lax.iota(jnp.int32, L) + (base + off)
            idx_bufs[g][pl.ds(k * L, L)] = rows * C + t

    copies = [
        pltpu.async_copy(lp_hbm.at[idx_bufs[g]], got_bufs[g], sem)
        for g in range(NG)
    ]
    for cp in copies:
        cp.wait()

    acc = jnp.zeros((L,), jnp.float32)
    for g in range(NG):
        for k in range(G // L):
            acc = acc + got_bufs[g][pl.ds(k * L, L)]
    my_v[...] = acc * (-1.0 / N)

    pltpu.sync_copy(my_v, out_hbm.at[wid])


@jax.jit
def _nll_sc(lp_flat, tgt):
    mesh = plsc.VectorSubcoreMesh(core_axis_name="c", subcore_axis_name="s")
    run = pl.kernel(
        _nll_body,
        mesh=mesh,
        out_type=jax.ShapeDtypeStruct((NW, L), jnp.float32),
        scratch_types=[
            pltpu.VMEM((RPW,), jnp.int32),
            pltpu.VMEM((G,), jnp.int32),
            pltpu.VMEM((G,), jnp.int32),
            pltpu.VMEM((G,), jnp.int32),
            pltpu.VMEM((G,), jnp.int32),
            pltpu.VMEM((G,), jnp.float32),
            pltpu.VMEM((G,), jnp.float32),
            pltpu.VMEM((G,), jnp.float32),
            pltpu.VMEM((G,), jnp.float32),
            pltpu.VMEM((L,), jnp.float32),
            pltpu.VMEM_SHARED((NS, L), jnp.float32),
            pltpu.SemaphoreType.DMA,
        ],
    )
    return run(lp_flat, tgt)


def kernel(logprob, target):
    lp_flat = logprob.reshape(-1)
    tgt = target.astype(jnp.int32)
    parts = _nll_sc(lp_flat, tgt)
    return jnp.sum(parts)


# speedup vs baseline: 1.3844x; 1.3844x over previous
import jax
import jax.numpy as jnp
from jax import lax
from jax.experimental import pallas as pl
from jax.experimental.pallas import tpu as pltpu
from jax.experimental.pallas import tpu_sc as plsc

N = 16384
C = 1000
L = 16
NC = 2
NS = 16
NW = NC * NS
RPW = N // NW
CH = 16
NCH = RPW // CH


def _nll_body(lp_hbm, tgt_hbm, out_hbm, tgt_v, buf_a, buf_b, my_v,
              sem_a, sem_b):
    cid = lax.axis_index("c")
    sid = lax.axis_index("s")
    wid = cid * NS + sid
    base = wid * RPW

    pltpu.sync_copy(tgt_hbm.at[pl.ds(base, RPW)], tgt_v)

    bufs = (buf_a, buf_b)
    sems = (sem_a, sem_b)
    copies = [None, None]
    copies[0] = pltpu.async_copy(
        lp_hbm.at[pl.ds(base, CH), :], buf_a, sem_a)

    rows = lax.iota(jnp.int32, L)
    acc = jnp.zeros((L,), jnp.float32)
    for i in range(NCH):
        cur = i % 2
        nxt = 1 - cur
        if i + 1 < NCH:
            copies[nxt] = pltpu.async_copy(
                lp_hbm.at[pl.ds(base + (i + 1) * CH, CH), :],
                bufs[nxt], sems[nxt])
        copies[cur].wait()
        t = tgt_v[pl.ds(i * CH, L)]
        acc = acc + plsc.load_gather(bufs[cur], [rows, t])
    my_v[...] = acc * (-1.0 / N)
    pltpu.sync_copy(my_v, out_hbm.at[wid])


@jax.jit
def _nll_sc(lp, tgt):
    mesh = plsc.VectorSubcoreMesh(core_axis_name="c", subcore_axis_name="s")
    run = pl.kernel(
        _nll_body,
        mesh=mesh,
        out_type=jax.ShapeDtypeStruct((NW, L), jnp.float32),
        scratch_types=[
            pltpu.VMEM((RPW,), jnp.int32),
            pltpu.VMEM((CH, C), jnp.float32),
            pltpu.VMEM((CH, C), jnp.float32),
            pltpu.VMEM((L,), jnp.float32),
            pltpu.SemaphoreType.DMA,
            pltpu.SemaphoreType.DMA,
        ],
        compiler_params=pltpu.CompilerParams(
            use_tc_tiling_on_sc=True, needs_layout_passes=False),
    )
    return run(lp, tgt)


def kernel(logprob, target):
    tgt = target.astype(jnp.int32)
    parts = _nll_sc(logprob, tgt)
    return jnp.sum(parts)


# speedup vs baseline: 1.3882x; 1.0027x over previous
import jax
import jax.numpy as jnp
from jax import lax
from jax.experimental import pallas as pl
from jax.experimental.pallas import tpu as pltpu
from jax.experimental.pallas import tpu_sc as plsc

N = 16384
C = 1000
L = 16
NC = 2
NS = 16
NW = NC * NS
RPW = N // NW
CH = 16
NCH = RPW // CH


def _nll_body(lp_hbm, tgt_hbm, out_hbm, tgt_v, buf_a, buf_b, my_v,
              sem_a, sem_b):
    cid = lax.axis_index("c")
    sid = lax.axis_index("s")
    wid = cid * NS + sid
    base = wid * RPW

    pltpu.sync_copy(tgt_hbm.at[pl.ds(base, RPW)], tgt_v)

    bufs = (buf_a, buf_b)
    sems = (sem_a, sem_b)
    last = base + RPW - CH

    def start(chunk, b):
        off = jnp.minimum(base + chunk * CH, last)
        pltpu.async_copy(lp_hbm.at[pl.ds(off, CH), :], bufs[b], sems[b])

    def drain(b):
        pltpu.make_async_copy(
            lp_hbm.at[pl.ds(base, CH), :], bufs[b], sems[b]).wait()

    start(0, 0)
    start(1, 1)

    rows = lax.iota(jnp.int32, L)

    def step(i, acc):
        for b in range(2):
            chunk = 2 * i + b
            drain(b)
            t = tgt_v[pl.ds(chunk * CH, L)]
            acc = acc + plsc.load_gather(bufs[b], [rows, t])
            start(chunk + 2, b)
        return acc

    acc = lax.fori_loop(0, NCH // 2 - 1, step, jnp.zeros((L,), jnp.float32))
    for b in range(2):
        chunk = NCH - 2 + b
        drain(b)
        t = tgt_v[pl.ds(chunk * CH, L)]
        acc = acc + plsc.load_gather(bufs[b], [rows, t])

    my_v[...] = acc * (-1.0 / N)
    pltpu.sync_copy(my_v, out_hbm.at[wid])


@jax.jit
def _nll_sc(lp, tgt):
    mesh = plsc.VectorSubcoreMesh(core_axis_name="c", subcore_axis_name="s")
    run = pl.kernel(
        _nll_body,
        mesh=mesh,
        out_type=jax.ShapeDtypeStruct((NW, L), jnp.float32),
        scratch_types=[
            pltpu.VMEM((RPW,), jnp.int32),
            pltpu.VMEM((CH, C), jnp.float32),
            pltpu.VMEM((CH, C), jnp.float32),
            pltpu.VMEM((L,), jnp.float32),
            pltpu.SemaphoreType.DMA,
            pltpu.SemaphoreType.DMA,
        ],
        compiler_params=pltpu.CompilerParams(
            use_tc_tiling_on_sc=True, needs_layout_passes=False),
    )
    return run(lp, tgt)


def kernel(logprob, target):
    tgt = target.astype(jnp.int32)
    parts = _nll_sc(logprob, tgt)
    return jnp.sum(parts)


# speedup vs baseline: 3.2623x; 2.3500x over previous
import jax
import jax.numpy as jnp
from jax import lax
from jax.experimental import pallas as pl
from jax.experimental.pallas import tpu as pltpu
from jax.experimental.pallas import tpu_sc as plsc

N = 16384
C = 1000
L = 16
NC = 2
NS = 16
NW = NC * NS
SPW = N // NW
CH = 128
NCH = SPW // CH


def _nll_body(lpt_hbm, tgt_hbm, out_hbm, tgt_v, buf, my_v, sem):
    cid = lax.axis_index("c")
    sid = lax.axis_index("s")
    wid = cid * NS + sid
    base = wid * SPW

    pltpu.sync_copy(tgt_hbm.at[pl.ds(base, SPW)], tgt_v)

    cols = lax.iota(jnp.int32, L)
    acc = jnp.zeros((L,), jnp.float32)
    for i in range(NCH):
        pltpu.async_copy(
            lpt_hbm.at[:, pl.ds(base + i * CH, CH)], buf, sem).wait()
        for k in range(CH // L):
            t = tgt_v[pl.ds(i * CH + k * L, L)]
            acc = acc + plsc.load_gather(buf, [t, cols + k * L])
    my_v[...] = acc * (-1.0 / N)
    pltpu.sync_copy(my_v, out_hbm.at[wid])


@jax.jit
def _nll_sc(lpt, tgt):
    mesh = plsc.VectorSubcoreMesh(core_axis_name="c", subcore_axis_name="s")
    run = pl.kernel(
        _nll_body,
        mesh=mesh,
        out_type=jax.ShapeDtypeStruct((NW, L), jnp.float32),
        scratch_types=[
            pltpu.VMEM((SPW,), jnp.int32),
            pltpu.VMEM((C, CH), jnp.float32),
            pltpu.VMEM((L,), jnp.float32),
            pltpu.SemaphoreType.DMA,
        ],
        compiler_params=pltpu.CompilerParams(
            use_tc_tiling_on_sc=True, needs_layout_passes=False),
    )
    return run(lpt, tgt)


def kernel(logprob, target):
    tgt = target.astype(jnp.int32)
    parts = _nll_sc(logprob.T, tgt)
    return jnp.sum(parts)


# speedup vs baseline: 6.5044x; 1.9938x over previous
import jax
import jax.numpy as jnp
from jax import lax
from jax.experimental import pallas as pl
from jax.experimental.pallas import tpu as pltpu
from jax.experimental.pallas import tpu_sc as plsc

N = 16384
C = 1000
L = 16
NC = 2
NS = 16
NW = NC * NS
SPW = N // NW
G = 128
NG = SPW // G


def _nll_body(lp_hbm, tgt_hbm, out_hbm,
              tgt_v, idx0, idx1, idx2, idx3, got0, got1, got2, got3,
              my_v, sem):
    idx_bufs = (idx0, idx1, idx2, idx3)
    got_bufs = (got0, got1, got2, got3)
    cid = lax.axis_index("c")
    sid = lax.axis_index("s")
    wid = cid * NS + sid
    base = wid * SPW

    pltpu.sync_copy(tgt_hbm.at[pl.ds(base, SPW)], tgt_v)

    lanes = lax.iota(jnp.int32, L)
    for g in range(NG):
        for k in range(G // L):
            off = g * G + k * L
            t = tgt_v[pl.ds(off, L)]
            n = lanes + (base + off)
            idx_bufs[g][pl.ds(k * L, L)] = (
                ((t >> 3) * 128 + (n >> 7)) * 1024 + (t & 7) * 128
                + (n & 127))

    copies = [
        pltpu.async_copy(lp_hbm.at[idx_bufs[g]], got_bufs[g], sem)
        for g in range(NG)
    ]
    for cp in copies:
        cp.wait()

    acc = jnp.zeros((L,), jnp.float32)
    for g in range(NG):
        for k in range(G // L):
            acc = acc + got_bufs[g][pl.ds(k * L, L)]
    my_v[...] = acc * (-1.0 / N)
    pltpu.sync_copy(my_v, out_hbm.at[wid])


@jax.jit
def _nll_sc(lp_lin, tgt):
    mesh = plsc.VectorSubcoreMesh(core_axis_name="c", subcore_axis_name="s")
    run = pl.kernel(
        _nll_body,
        mesh=mesh,
        out_type=jax.ShapeDtypeStruct((NW, L), jnp.float32),
        scratch_types=[
            pltpu.VMEM((SPW,), jnp.int32),
            pltpu.VMEM((G,), jnp.int32),
            pltpu.VMEM((G,), jnp.int32),
            pltpu.VMEM((G,), jnp.int32),
            pltpu.VMEM((G,), jnp.int32),
            pltpu.VMEM((G,), jnp.float32),
            pltpu.VMEM((G,), jnp.float32),
            pltpu.VMEM((G,), jnp.float32),
            pltpu.VMEM((G,), jnp.float32),
            pltpu.VMEM((L,), jnp.float32),
            pltpu.SemaphoreType.DMA,
        ],
    )
    return run(lp_lin, tgt)


def kernel(logprob, target):
    lp_lin = (logprob.T.reshape(C // 8, 8, N // 128, 128)
              .transpose(0, 2, 1, 3).reshape(-1))
    tgt = target.astype(jnp.int32)
    parts = _nll_sc(lp_lin, tgt)
    return jnp.sum(parts)


# speedup vs baseline: 6.6272x; 1.0189x over previous
import jax
import jax.numpy as jnp
from jax import lax
from jax.experimental import pallas as pl
from jax.experimental.pallas import tpu as pltpu
from jax.experimental.pallas import tpu_sc as plsc

N = 16384
C = 1000
L = 16
NC = 2
NS = 16
NW = NC * NS
SPW = N // NW
G = 128
NG = SPW // G


def _nll_body(lp_hbm, tgt_hbm, out_hbm,
              tgt_v, idx0, idx1, idx2, idx3, got0, got1, got2, got3,
              my_v, sem):
    idx_bufs = (idx0, idx1, idx2, idx3)
    got_bufs = (got0, got1, got2, got3)
    cid = lax.axis_index("c")
    sid = lax.axis_index("s")
    wid = cid * NS + sid
    base = wid * SPW

    pltpu.sync_copy(tgt_hbm.at[pl.ds(base, SPW)], tgt_v)

    lanes = lax.iota(jnp.int32, L)

    for g in range(NG):
        def idx_step(k, _, g=g):
            off = g * G + k * L
            t = tgt_v[pl.ds(off, L)]
            n = lanes + (base + off)
            idx_bufs[g][pl.ds(k * L, L)] = (
                ((t >> 3) * 128 + (n >> 7)) * 1024 + (t & 7) * 128
                + (n & 127))
            return 0
        lax.fori_loop(0, G // L, idx_step, 0)
        pltpu.async_copy(lp_hbm.at[idx_bufs[g]], got_bufs[g], sem)

    acc = jnp.zeros((L,), jnp.float32)
    for g in range(NG):
        pltpu.make_async_copy(
            lp_hbm.at[idx_bufs[g]], got_bufs[g], sem).wait()

        def sum_step(k, a, g=g):
            return a + got_bufs[g][pl.ds(k * L, L)]
        acc = lax.fori_loop(0, G // L, sum_step, acc)
    my_v[...] = acc * (-1.0 / N)
    pltpu.sync_copy(my_v, out_hbm.at[wid])


@jax.jit
def _nll_sc(lp_lin, tgt):
    mesh = plsc.VectorSubcoreMesh(core_axis_name="c", subcore_axis_name="s")
    run = pl.kernel(
        _nll_body,
        mesh=mesh,
        out_type=jax.ShapeDtypeStruct((NW, L), jnp.float32),
        scratch_types=[
            pltpu.VMEM((SPW,), jnp.int32),
            pltpu.VMEM((G,), jnp.int32),
            pltpu.VMEM((G,), jnp.int32),
            pltpu.VMEM((G,), jnp.int32),
            pltpu.VMEM((G,), jnp.int32),
            pltpu.VMEM((G,), jnp.float32),
            pltpu.VMEM((G,), jnp.float32),
            pltpu.VMEM((G,), jnp.float32),
            pltpu.VMEM((G,), jnp.float32),
            pltpu.VMEM((L,), jnp.float32),
            pltpu.SemaphoreType.DMA,
        ],
    )
    return run(lp_lin, tgt)


def kernel(logprob, target):
    lp_lin = (logprob.T.reshape(C // 8, 8, N // 128, 128)
              .transpose(0, 2, 1, 3).reshape(-1))
    tgt = target.astype(jnp.int32)
    parts = _nll_sc(lp_lin, tgt)
    return jnp.sum(parts)


# speedup vs baseline: 6.9285x; 1.0454x over previous
import jax
import jax.numpy as jnp
from jax import lax
from jax.experimental import pallas as pl
from jax.experimental.pallas import tpu as pltpu
from jax.experimental.pallas import tpu_sc as plsc

N = 16384
C = 1000
L = 16
NS = 16
SPW = N // NS
G = 128
NG = SPW // G


def _nll_body(lp_hbm, tgt_hbm, out_hbm, refs):
    (tgt_v, idx_bufs, got_bufs, my_v, comb_v, shared, sem) = refs
    sid = lax.axis_index("s")
    base = sid * SPW

    pltpu.sync_copy(tgt_hbm.at[pl.ds(base, SPW)], tgt_v)

    lanes = lax.iota(jnp.int32, L)
    for g in range(NG):
        def idx_step(k, _, g=g):
            off = g * G + k * L
            t = tgt_v[pl.ds(off, L)]
            n = lanes + (base + off)
            idx_bufs[g][pl.ds(k * L, L)] = (
                ((t >> 3) * 128 + (n >> 7)) * 1024 + (t & 7) * 128
                + (n & 127))
            return 0
        lax.fori_loop(0, G // L, idx_step, 0)
        pltpu.async_copy(lp_hbm.at[idx_bufs[g]], got_bufs[g], sem)

    acc = jnp.zeros((L,), jnp.float32)
    for g in range(NG):
        pltpu.make_async_copy(
            lp_hbm.at[idx_bufs[g]], got_bufs[g], sem).wait()

        def sum_step(k, a, g=g):
            return a + got_bufs[g][pl.ds(k * L, L)]
        acc = lax.fori_loop(0, G // L, sum_step, acc)
    my_v[...] = acc

    pltpu.sync_copy(my_v, shared.at[sid])
    plsc.subcore_barrier()

    @pl.when(sid == 0)
    def _():
        pltpu.sync_copy(shared, comb_v)
        tot = jnp.zeros((L,), jnp.float32)
        for i in range(NS):
            tot = tot + comb_v[i]
        loss = jnp.sum(tot) * (-1.0 / N)
        my_v[...] = jnp.full((L,), loss, jnp.float32)
        pltpu.sync_copy(my_v, out_hbm)


@jax.jit
def _nll_sc(lp_lin, tgt):
    mesh = plsc.VectorSubcoreMesh(
        core_axis_name="c", subcore_axis_name="s", num_cores=1)
    run = pl.kernel(
        _nll_body,
        mesh=mesh,
        out_type=jax.ShapeDtypeStruct((L,), jnp.float32),
        scratch_types=[(
            pltpu.VMEM((SPW,), jnp.int32),
            tuple(pltpu.VMEM((G,), jnp.int32) for _ in range(NG)),
            tuple(pltpu.VMEM((G,), jnp.float32) for _ in range(NG)),
            pltpu.VMEM((L,), jnp.float32),
            pltpu.VMEM((NS, L), jnp.float32),
            pltpu.VMEM_SHARED((NS, L), jnp.float32),
            pltpu.SemaphoreType.DMA,
        )],
        compiler_params=pltpu.CompilerParams(needs_layout_passes=False),
    )
    return run(lp_lin, tgt)


def kernel(logprob, target):
    lp_lin = (logprob.T.reshape(C // 8, 8, N // 128, 128)
              .transpose(0, 2, 1, 3).reshape(-1))
    tgt = target.astype(jnp.int32)
    return _nll_sc(lp_lin, tgt)[0]
